# baseline (speedup 1.0000x reference)
import functools

import jax
import jax.numpy as jnp
from jax import lax
from jax.experimental import pallas as pl
from jax.experimental.pallas import tpu as pltpu
from jax.experimental.pallas import tpu_sc as plsc

_N = 10000
_E = 320000
_D = 128
_DE = 16
_H = 128
_G = 64

_NC = 2
_NS = 16
_EPT = _E // (_NC * _NS)
_CH = 80
_NCH = _EPT // _CH
_NP = 10240
_RPT = _NP // _NS
_NZ = _RPT // _CH

_R = 2000
_NB = _N // _R
_PB = 32000

_F32 = jnp.float32
_HI = lax.Precision.HIGHEST

_SC_MESH = plsc.VectorSubcoreMesh(core_axis_name="c", subcore_axis_name="s")
_SC_PARAMS = pltpu.CompilerParams(use_tc_tiling_on_sc=False)


def _prep_body(eir, sr, dr):
    sr[...] = eir[0]
    dr[...] = eir[1]


_prep = pl.pallas_call(
    _prep_body,
    out_shape=[jax.ShapeDtypeStruct((_E,), jnp.int32),
               jax.ShapeDtypeStruct((_E,), jnp.int32)],
)


def _sca_body(x_hbm, src_hbm, dst_hbm, aggx_out,
              src_v, d0, d1, d2, r0, r1, r2, aggx_sh,
              sd0, sd1, sd2, sg0, sg1, sg2):
    cc = lax.axis_index("c")
    ss = lax.axis_index("s")
    eb = (cc * _NS + ss) * _EPT
    rb = ss * _RPT
    sets = ((d0, r0, sd0, sg0), (d1, r1, sd1, sg1), (d2, r2, sd2, sg2))

    def _zr(i, _):
        r0[i // 8, pl.ds((i % 8) * 16, 16)] = jnp.zeros((16,), _F32)
        return 0

    lax.fori_loop(0, _CH * 8, _zr, 0)

    def _zs(k, _):
        pltpu.sync_copy(r0, aggx_sh.at[pl.ds(rb + k * _CH, _CH)])
        return 0

    lax.fori_loop(0, _NZ, _zs, 0)
    plsc.subcore_barrier()

    pltpu.sync_copy(src_hbm.at[pl.ds(eb, _EPT)], src_v)

    def _start(c, k):
        d, r, sd, sg = sets[k]
        pltpu.async_copy(dst_hbm.at[pl.ds(eb + c * _CH, _CH)], d, sd)
        pltpu.async_copy(x_hbm.at[src_v.at[pl.ds(c * _CH, _CH)]], r, sg)

    def _drain(k):
        d, r, sd, sg = sets[k]
        pltpu.make_async_copy(x_hbm.at[pl.ds(0, _CH)], r, sg).wait()
        pltpu.make_async_copy(dst_hbm.at[pl.ds(0, _CH)], d, sd).wait()
        pltpu.sync_copy(r, aggx_sh.at[d], add=True)

    _start(0, 0)
    _start(1, 1)
    _start(2, 2)

    def _body(q, _):
        c0 = 3 * q
        for k in range(3):
            _drain(k)

            @pl.when(c0 + k + 3 < _NCH)
            def _(c=c0 + k + 3, k=k):
                _start(c, k)

        return 0

    lax.fori_loop(0, _NCH // 3, _body, 0)
    _drain(0)
    _drain(1)
    plsc.subcore_barrier()

    def _wb(k, _):
        r = rb + k * _CH
        pltpu.sync_copy(aggx_sh.at[pl.ds(r, _CH)], r0)
        pltpu.sync_copy(r0, aggx_out.at[cc, pl.ds(r, _CH)])
        return 0

    lax.fori_loop(0, _NZ, _wb, 0)


_sc_aggx = functools.partial(
    pl.kernel,
    out_type=jax.ShapeDtypeStruct((_NC, _NP, _D), _F32),
    mesh=_SC_MESH,
    compiler_params=_SC_PARAMS,
    scratch_types=[
        pltpu.VMEM((_EPT,), jnp.int32),
        pltpu.VMEM((_CH,), jnp.int32),
        pltpu.VMEM((_CH,), jnp.int32),
        pltpu.VMEM((_CH,), jnp.int32),
        pltpu.VMEM((_CH, _D), _F32),
        pltpu.VMEM((_CH, _D), _F32),
        pltpu.VMEM((_CH, _D), _F32),
        pltpu.VMEM_SHARED((_NP, _D), _F32),
        pltpu.SemaphoreType.DMA,
        pltpu.SemaphoreType.DMA,
        pltpu.SemaphoreType.DMA,
        pltpu.SemaphoreType.DMA,
        pltpu.SemaphoreType.DMA,
        pltpu.SemaphoreType.DMA,
    ],
)(_sca_body)


_ST0 = 4960
_ST1 = 5040
_NCH0 = _ST0 // _CH
_NCH1 = _ST1 // _CH


def _scb_body(ea1_hbm, dst_hbm, aggx_hbm, agge_out,
              eav, d0, d1, e2d, agge_sh, sv, sd0, sd1):
    del aggx_hbm
    cc = lax.axis_index("c")
    ss = lax.axis_index("s")
    eb = (cc * _NS + ss) * _EPT
    rb = ss * _RPT

    def _ze(i, _):
        e2d[i, :] = jnp.zeros((16,), _F32)
        return 0

    lax.fori_loop(0, _CH, _ze, 0)

    def _zs(k, _):
        pltpu.sync_copy(e2d, agge_sh.at[pl.ds(rb + k * _CH, _CH)])
        return 0

    lax.fori_loop(0, _NZ, _zs, 0)
    plsc.subcore_barrier()

    dsets = ((d0, sd0), (d1, sd1))

    def _start_didx(c, k):
        d, sd = dsets[k]
        pltpu.async_copy(dst_hbm.at[pl.ds(eb + c * _CH, _CH)], d, sd)

    def _stage_load(base, size):
        for c in range(_DE):
            pltpu.async_copy(ea1_hbm.at[pl.ds(c * _E + eb + base, size)],
                             eav.at[pl.ds(c * size, size)], sv)
        for c in range(_DE):
            pltpu.make_async_copy(ea1_hbm.at[pl.ds(0, size)],
                                  eav.at[pl.ds(c * size, size)], sv).wait()

    def _chunk(off, size, k):
        d, sd = dsets[k]
        colbase = lax.iota(jnp.int32, 16) * size

        def _row(r, _):
            vals = plsc.load_gather(eav, [colbase + (off + r)])
            e2d[r, :] = vals
            return 0

        lax.fori_loop(0, _CH, _row, 0)
        pltpu.make_async_copy(dst_hbm.at[pl.ds(0, _CH)], d, sd).wait()
        pltpu.sync_copy(e2d, agge_sh.at[d], add=True)

    _stage_load(0, _ST0)
    _start_didx(0, 0)

    def _body0(j, _):
        c0 = 2 * j
        _start_didx(c0 + 1, 1)
        _chunk(c0 * _CH, _ST0, 0)
        _start_didx(c0 + 2, 0)
        _chunk((c0 + 1) * _CH, _ST0, 1)
        return 0

    lax.fori_loop(0, _NCH0 // 2, _body0, 0)
    _stage_load(_ST0, _ST1)

    def _body1(j, _):
        c0 = _NCH0 + 2 * j
        _start_didx(c0 + 1, 1)
        _chunk((c0 - _NCH0) * _CH, _ST1, 0)
        _start_didx(c0 + 2, 0)
        _chunk((c0 + 1 - _NCH0) * _CH, _ST1, 1)
        return 0

    lax.fori_loop(0, (_NCH1 - 1) // 2, _body1, 0)
    _chunk((_NCH1 - 1) * _CH, _ST1, 0)
    plsc.subcore_barrier()

    def _wb(k, _):
        r = rb + k * _CH
        pltpu.sync_copy(agge_sh.at[pl.ds(r, _CH)], e2d)
        pltpu.sync_copy(e2d, agge_out.at[cc, pl.ds(r, _CH)])
        return 0

    lax.fori_loop(0, _NZ, _wb, 0)


_sc_agge = functools.partial(
    pl.kernel,
    out_type=jax.ShapeDtypeStruct((_NC, _NP, _DE), _F32),
    mesh=_SC_MESH,
    compiler_params=pltpu.CompilerParams(use_tc_tiling_on_sc=False,
                                         needs_layout_passes=False),
    scratch_types=[
        pltpu.VMEM((_DE * _ST1,), _F32),
        pltpu.VMEM((_CH,), jnp.int32),
        pltpu.VMEM((_CH,), jnp.int32),
        pltpu.VMEM((_CH, _DE), _F32),
        pltpu.VMEM_SHARED((_NP, _DE), _F32),
        pltpu.SemaphoreType.DMA,
        pltpu.SemaphoreType.DMA,
        pltpu.SemaphoreType.DMA,
    ],
)(_scb_body)


def _tc_body(xr, a0r, a1r, e0r, e1r, br, wsr, wmr, wer, bmr, wpr, bpr,
             outr, gsumr, cntr):
    i = pl.program_id(0)

    @pl.when(i == 0)
    def _init():
        gsumr[...] = jnp.zeros_like(gsumr)
        cntr[...] = jnp.zeros_like(cntr)

    h = (jnp.dot(xr[...], wsr[...], precision=_HI, preferred_element_type=_F32)
         + jnp.dot(a0r[0] + a1r[0], wmr[...], precision=_HI,
                   preferred_element_type=_F32)
         + jnp.dot(e0r[0] + e1r[0], wer[...], precision=_HI,
                   preferred_element_type=_F32)
         + bmr[...])
    h = jnp.where(h > 0, h, 0.01 * h)

    oht = (br[0] == lax.broadcasted_iota(jnp.int32, (_G, _R), 0)).astype(_F32)
    gsumr[...] += jnp.dot(oht, h, precision=_HI, preferred_element_type=_F32)
    cntr[...] += jnp.dot(oht, jnp.ones((_R, _D), _F32), precision=_HI,
                         preferred_element_type=_F32)

    @pl.when(i == _NB - 1)
    def _fin():
        gmean = gsumr[...] / jnp.maximum(cntr[...], 1.0)
        n2 = jnp.sum(gmean * gmean, axis=1, keepdims=True)
        nrm = jnp.maximum(jnp.sqrt(n2), 1e-12)
        embs = (gmean / nrm).astype(jnp.bfloat16).astype(_F32)
        wp16 = wpr[...].astype(jnp.bfloat16).astype(_F32)
        outr[...] = jnp.sum(embs * wp16, axis=1, keepdims=True) + bpr[...]


_tc_head = pl.pallas_call(
    _tc_body,
    grid=(_NB,),
    in_specs=[
        pl.BlockSpec((_R, _D), lambda i: (i, 0)),
        pl.BlockSpec((1, _R, _D), lambda i: (0, i, 0)),
        pl.BlockSpec((1, _R, _D), lambda i: (1, i, 0)),
        pl.BlockSpec((1, _R, _DE), lambda i: (0, i, 0)),
        pl.BlockSpec((1, _R, _DE), lambda i: (1, i, 0)),
        pl.BlockSpec((1, 1, _R), lambda i: (i, 0, 0)),
        pl.BlockSpec((_D, _H), lambda i: (0, 0)),
        pl.BlockSpec((_D, _H), lambda i: (0, 0)),
        pl.BlockSpec((_DE, _H), lambda i: (0, 0)),
        pl.BlockSpec((1, _H), lambda i: (0, 0)),
        pl.BlockSpec((1, _H), lambda i: (0, 0)),
        pl.BlockSpec((1, 1), lambda i: (0, 0)),
    ],
    out_specs=pl.BlockSpec((_G, 1), lambda i: (0, 0)),
    out_shape=jax.ShapeDtypeStruct((_G, 1), _F32),
    scratch_shapes=[
        pltpu.VMEM((_G, _D), _F32),
        pltpu.VMEM((_G, _D), _F32),
    ],
)


def kernel(x, edge_index, edge_attr, batch, W_self, W_msg, W_edge, b_msg,
           Wp, bp):
    src, dst = _prep(edge_index)
    aggx = _sc_aggx(x, src, dst)
    agge = _sc_agge(edge_attr.T.reshape(_E * _DE), dst, aggx)
    return _tc_head(x, aggx, aggx, agge, agge, batch.reshape(_NB, 1, _R),
                    W_self, W_msg, W_edge, b_msg.reshape(1, _H),
                    Wp.reshape(1, _H), bp.reshape(1, 1))


# speedup vs baseline: 1.0422x; 1.0422x over previous
import functools

import jax
import jax.numpy as jnp
from jax import lax
from jax.experimental import pallas as pl
from jax.experimental.pallas import tpu as pltpu
from jax.experimental.pallas import tpu_sc as plsc

_N = 10000
_E = 320000
_D = 128
_DE = 16
_H = 128
_G = 64

_NC = 2
_NS = 16
_EPT = _E // (_NC * _NS)
_CH = 80
_NCH = _EPT // _CH
_NP = 10240
_RPT = _NP // _NS
_NZ = _RPT // _CH

_R = 2000
_NB = _N // _R
_PB = 32000

_F32 = jnp.float32
_HI = lax.Precision.HIGHEST

_SC_MESH = plsc.VectorSubcoreMesh(core_axis_name="c", subcore_axis_name="s")
_SC_PARAMS = pltpu.CompilerParams(use_tc_tiling_on_sc=False)


def _prep_body(eir, sr, dr):
    sr[...] = eir[0]
    dr[...] = eir[1]


_prep = pl.pallas_call(
    _prep_body,
    out_shape=[jax.ShapeDtypeStruct((_E,), jnp.int32),
               jax.ShapeDtypeStruct((_E,), jnp.int32)],
)


def _sca_body(x_hbm, src_hbm, dst_hbm, aggx_out,
              src_v, d0, d1, d2, r0, r1, r2, aggx_sh,
              sd0, sd1, sd2, sg0, sg1, sg2):
    cc = lax.axis_index("c")
    ss = lax.axis_index("s")
    eb = (cc * _NS + ss) * _EPT
    rb = ss * _RPT
    sets = ((d0, r0, sd0, sg0), (d1, r1, sd1, sg1), (d2, r2, sd2, sg2))

    def _zr(i, _):
        r0[i // 8, pl.ds((i % 8) * 16, 16)] = jnp.zeros((16,), _F32)
        return 0

    lax.fori_loop(0, _CH * 8, _zr, 0)

    def _zs(k, _):
        pltpu.sync_copy(r0, aggx_sh.at[pl.ds(rb + k * _CH, _CH)])
        return 0

    lax.fori_loop(0, _NZ, _zs, 0)
    plsc.subcore_barrier()

    pltpu.sync_copy(src_hbm.at[pl.ds(eb, _EPT)], src_v)

    def _start(c, k):
        d, r, sd, sg = sets[k]
        pltpu.async_copy(dst_hbm.at[pl.ds(eb + c * _CH, _CH)], d, sd)
        pltpu.async_copy(x_hbm.at[src_v.at[pl.ds(c * _CH, _CH)]], r, sg)

    def _drain(k):
        d, r, sd, sg = sets[k]
        pltpu.make_async_copy(x_hbm.at[pl.ds(0, _CH)], r, sg).wait()
        pltpu.make_async_copy(dst_hbm.at[pl.ds(0, _CH)], d, sd).wait()
        pltpu.sync_copy(r, aggx_sh.at[d], add=True)

    _start(0, 0)
    _start(1, 1)
    _start(2, 2)

    def _body(q, _):
        c0 = 3 * q
        for k in range(3):
            _drain(k)

            @pl.when(c0 + k + 3 < _NCH)
            def _(c=c0 + k + 3, k=k):
                _start(c, k)

        return 0

    lax.fori_loop(0, _NCH // 3, _body, 0)
    _drain(0)
    _drain(1)
    plsc.subcore_barrier()

    def _wb(k, _):
        r = rb + k * _CH
        pltpu.sync_copy(aggx_sh.at[pl.ds(r, _CH)], r0)
        pltpu.sync_copy(r0, aggx_out.at[cc, pl.ds(r, _CH)])
        return 0

    lax.fori_loop(0, _NZ, _wb, 0)


_sc_aggx = functools.partial(
    pl.kernel,
    out_type=jax.ShapeDtypeStruct((_NC, _NP, _D), _F32),
    mesh=_SC_MESH,
    compiler_params=_SC_PARAMS,
    scratch_types=[
        pltpu.VMEM((_EPT,), jnp.int32),
        pltpu.VMEM((_CH,), jnp.int32),
        pltpu.VMEM((_CH,), jnp.int32),
        pltpu.VMEM((_CH,), jnp.int32),
        pltpu.VMEM((_CH, _D), _F32),
        pltpu.VMEM((_CH, _D), _F32),
        pltpu.VMEM((_CH, _D), _F32),
        pltpu.VMEM_SHARED((_NP, _D), _F32),
        pltpu.SemaphoreType.DMA,
        pltpu.SemaphoreType.DMA,
        pltpu.SemaphoreType.DMA,
        pltpu.SemaphoreType.DMA,
        pltpu.SemaphoreType.DMA,
        pltpu.SemaphoreType.DMA,
    ],
)(_sca_body)


_ST0 = 4960
_ST1 = 5040
_NCH0 = _ST0 // _CH
_NCH1 = _ST1 // _CH


def _scb_body(ea1_hbm, dst_hbm, aggx_hbm, agge_out,
              eav, d0, d1, e2d, e2b, agge_sh, sv, sd0, sd1, sc0, sc1):
    del aggx_hbm
    cc = lax.axis_index("c")
    ss = lax.axis_index("s")
    eb = (cc * _NS + ss) * _EPT
    rb = ss * _RPT

    def _ze(i, _):
        e2d[i, :] = jnp.zeros((16,), _F32)
        return 0

    lax.fori_loop(0, _CH, _ze, 0)

    def _zs(k, _):
        pltpu.sync_copy(e2d, agge_sh.at[pl.ds(rb + k * _CH, _CH)])
        return 0

    lax.fori_loop(0, _NZ, _zs, 0)
    plsc.subcore_barrier()

    dsets = ((d0, sd0, e2d, sc0), (d1, sd1, e2b, sc1))

    def _start_didx(c, k):
        d = dsets[k][0]
        sd = dsets[k][1]
        pltpu.async_copy(dst_hbm.at[pl.ds(eb + c * _CH, _CH)], d, sd)

    def _stage_load(base, size):
        for c in range(_DE):
            pltpu.async_copy(ea1_hbm.at[pl.ds(c * _E + eb + base, size)],
                             eav.at[pl.ds(c * size, size)], sv)
        for c in range(_DE):
            pltpu.make_async_copy(ea1_hbm.at[pl.ds(0, size)],
                                  eav.at[pl.ds(c * size, size)], sv).wait()

    def _chunk(off, size, k, wait_prev):
        d, sd, e2, sc = dsets[k]
        colbase = lax.iota(jnp.int32, 16) * size

        @pl.when(wait_prev)
        def _():
            pltpu.make_async_copy(e2, agge_sh.at[d], sc).wait()

        def _row(r, _):
            e2[r, :] = plsc.load_gather(eav, [colbase + (off + r)])
            return 0

        lax.fori_loop(0, _CH, _row, 0, unroll=16)
        pltpu.make_async_copy(dst_hbm.at[pl.ds(0, _CH)], d, sd).wait()
        pltpu.async_copy(e2, agge_sh.at[d], sc, add=True)

    _stage_load(0, _ST0)
    _start_didx(0, 0)

    def _body0(j, _):
        c0 = 2 * j
        _start_didx(c0 + 1, 1)
        _chunk(c0 * _CH, _ST0, 0, j > 0)
        _start_didx(c0 + 2, 0)
        _chunk((c0 + 1) * _CH, _ST0, 1, j > 0)
        return 0

    lax.fori_loop(0, _NCH0 // 2, _body0, 0)
    _stage_load(_ST0, _ST1)

    def _body1(j, _):
        c0 = _NCH0 + 2 * j
        _start_didx(c0 + 1, 1)
        _chunk((c0 - _NCH0) * _CH, _ST1, 0, True)
        _start_didx(c0 + 2, 0)
        _chunk((c0 + 1 - _NCH0) * _CH, _ST1, 1, True)
        return 0

    lax.fori_loop(0, (_NCH1 - 1) // 2, _body1, 0)
    _chunk((_NCH1 - 1) * _CH, _ST1, 0, True)
    pltpu.make_async_copy(e2d, agge_sh.at[d0], sc0).wait()
    pltpu.make_async_copy(e2b, agge_sh.at[d1], sc1).wait()
    plsc.subcore_barrier()

    def _wb(k, _):
        r = rb + k * _CH
        pltpu.sync_copy(agge_sh.at[pl.ds(r, _CH)], e2d)
        pltpu.sync_copy(e2d, agge_out.at[cc, pl.ds(r, _CH)])
        return 0

    lax.fori_loop(0, _NZ, _wb, 0)


_sc_agge = functools.partial(
    pl.kernel,
    out_type=jax.ShapeDtypeStruct((_NC, _NP, _DE), _F32),
    mesh=_SC_MESH,
    compiler_params=pltpu.CompilerParams(use_tc_tiling_on_sc=False,
                                         needs_layout_passes=False),
    scratch_types=[
        pltpu.VMEM((_DE * _ST1,), _F32),
        pltpu.VMEM((_CH,), jnp.int32),
        pltpu.VMEM((_CH,), jnp.int32),
        pltpu.VMEM((_CH, _DE), _F32),
        pltpu.VMEM((_CH, _DE), _F32),
        pltpu.VMEM_SHARED((_NP, _DE), _F32),
        pltpu.SemaphoreType.DMA,
        pltpu.SemaphoreType.DMA,
        pltpu.SemaphoreType.DMA,
        pltpu.SemaphoreType.DMA,
        pltpu.SemaphoreType.DMA,
    ],
)(_scb_body)


def _tc_body(xr, a0r, a1r, e0r, e1r, br, wsr, wmr, wer, bmr, wpr, bpr,
             outr, gsumr, cntr):
    i = pl.program_id(0)

    @pl.when(i == 0)
    def _init():
        gsumr[...] = jnp.zeros_like(gsumr)
        cntr[...] = jnp.zeros_like(cntr)

    h = (jnp.dot(xr[...], wsr[...], precision=_HI, preferred_element_type=_F32)
         + jnp.dot(a0r[0] + a1r[0], wmr[...], precision=_HI,
                   preferred_element_type=_F32)
         + jnp.dot(e0r[0] + e1r[0], wer[...], precision=_HI,
                   preferred_element_type=_F32)
         + bmr[...])
    h = jnp.where(h > 0, h, 0.01 * h)

    oht = (br[0] == lax.broadcasted_iota(jnp.int32, (_G, _R), 0)).astype(_F32)
    gsumr[...] += jnp.dot(oht, h, precision=_HI, preferred_element_type=_F32)
    cntr[...] += jnp.dot(oht, jnp.ones((_R, _D), _F32), precision=_HI,
                         preferred_element_type=_F32)

    @pl.when(i == _NB - 1)
    def _fin():
        gmean = gsumr[...] / jnp.maximum(cntr[...], 1.0)
        n2 = jnp.sum(gmean * gmean, axis=1, keepdims=True)
        nrm = jnp.maximum(jnp.sqrt(n2), 1e-12)
        embs = (gmean / nrm).astype(jnp.bfloat16).astype(_F32)
        wp16 = wpr[...].astype(jnp.bfloat16).astype(_F32)
        outr[...] = jnp.sum(embs * wp16, axis=1, keepdims=True) + bpr[...]


_tc_head = pl.pallas_call(
    _tc_body,
    grid=(_NB,),
    in_specs=[
        pl.BlockSpec((_R, _D), lambda i: (i, 0)),
        pl.BlockSpec((1, _R, _D), lambda i: (0, i, 0)),
        pl.BlockSpec((1, _R, _D), lambda i: (1, i, 0)),
        pl.BlockSpec((1, _R, _DE), lambda i: (0, i, 0)),
        pl.BlockSpec((1, _R, _DE), lambda i: (1, i, 0)),
        pl.BlockSpec((1, 1, _R), lambda i: (i, 0, 0)),
        pl.BlockSpec((_D, _H), lambda i: (0, 0)),
        pl.BlockSpec((_D, _H), lambda i: (0, 0)),
        pl.BlockSpec((_DE, _H), lambda i: (0, 0)),
        pl.BlockSpec((1, _H), lambda i: (0, 0)),
        pl.BlockSpec((1, _H), lambda i: (0, 0)),
        pl.BlockSpec((1, 1), lambda i: (0, 0)),
    ],
    out_specs=pl.BlockSpec((_G, 1), lambda i: (0, 0)),
    out_shape=jax.ShapeDtypeStruct((_G, 1), _F32),
    scratch_shapes=[
        pltpu.VMEM((_G, _D), _F32),
        pltpu.VMEM((_G, _D), _F32),
    ],
)


def kernel(x, edge_index, edge_attr, batch, W_self, W_msg, W_edge, b_msg,
           Wp, bp):
    src, dst = _prep(edge_index)
    aggx = _sc_aggx(x, src, dst)
    agge = _sc_agge(edge_attr.T.reshape(_E * _DE), dst, aggx)
    return _tc_head(x, aggx, aggx, agge, agge, batch.reshape(_NB, 1, _R),
                    W_self, W_msg, W_edge, b_msg.reshape(1, _H),
                    Wp.reshape(1, _H), bp.reshape(1, 1))


# speedup vs baseline: 1.2063x; 1.1575x over previous
import functools

import jax
import jax.numpy as jnp
from jax import lax
from jax.experimental import pallas as pl
from jax.experimental.pallas import tpu as pltpu
from jax.experimental.pallas import tpu_sc as plsc

_N = 10000
_E = 320000
_D = 128
_DE = 16
_H = 128
_G = 64

_NC = 2
_NS = 16
_EPT = _E // (_NC * _NS)
_CH = 80
_NCH = _EPT // _CH
_NP = 10240
_RPT = _NP // _NS
_NZ = _RPT // _CH

_R = 2000
_NB = _N // _R
_PB = 32000

_F32 = jnp.float32
_HI = lax.Precision.HIGHEST

_SC_MESH = plsc.VectorSubcoreMesh(core_axis_name="c", subcore_axis_name="s")
_SC_PARAMS = pltpu.CompilerParams(use_tc_tiling_on_sc=False)


def _prep_body(eir, sr, dr):
    sr[...] = eir[0]
    dr[...] = eir[1]


_prep = pl.pallas_call(
    _prep_body,
    out_shape=[jax.ShapeDtypeStruct((_E,), jnp.int32),
               jax.ShapeDtypeStruct((_E,), jnp.int32)],
)


def _sca_body(x_hbm, src_hbm, dst_hbm, aggx_out,
              src_v, d0, d1, d2, r0, r1, r2, aggx_sh,
              sd0, sd1, sd2, sg0, sg1, sg2):
    cc = lax.axis_index("c")
    ss = lax.axis_index("s")
    eb = (cc * _NS + ss) * _EPT
    rb = ss * _RPT
    sets = ((d0, r0, sd0, sg0), (d1, r1, sd1, sg1), (d2, r2, sd2, sg2))

    def _zr(i, _):
        r0[i // 8, pl.ds((i % 8) * 16, 16)] = jnp.zeros((16,), _F32)
        return 0

    lax.fori_loop(0, _CH * 8, _zr, 0)

    def _zs(k, _):
        pltpu.sync_copy(r0, aggx_sh.at[pl.ds(rb + k * _CH, _CH)])
        return 0

    lax.fori_loop(0, _NZ, _zs, 0)
    plsc.subcore_barrier()

    pltpu.sync_copy(src_hbm.at[pl.ds(eb, _EPT)], src_v)

    def _start(c, k):
        d, r, sd, sg = sets[k]
        pltpu.async_copy(dst_hbm.at[pl.ds(eb + c * _CH, _CH)], d, sd)
        pltpu.async_copy(x_hbm.at[src_v.at[pl.ds(c * _CH, _CH)]], r, sg)

    def _drain(k):
        d, r, sd, sg = sets[k]
        pltpu.make_async_copy(x_hbm.at[pl.ds(0, _CH)], r, sg).wait()
        pltpu.make_async_copy(dst_hbm.at[pl.ds(0, _CH)], d, sd).wait()
        pltpu.sync_copy(r, aggx_sh.at[d], add=True)

    _start(0, 0)
    _start(1, 1)
    _start(2, 2)

    def _body(q, _):
        c0 = 3 * q
        for k in range(3):
            _drain(k)

            @pl.when(c0 + k + 3 < _NCH)
            def _(c=c0 + k + 3, k=k):
                _start(c, k)

        return 0

    lax.fori_loop(0, _NCH // 3, _body, 0)
    _drain(0)
    _drain(1)
    plsc.subcore_barrier()

    def _wb(k, _):
        r = rb + k * _CH
        pltpu.sync_copy(aggx_sh.at[pl.ds(r, _CH)], r0)
        pltpu.sync_copy(r0, aggx_out.at[cc, pl.ds(r, _CH)])
        return 0

    lax.fori_loop(0, _NZ, _wb, 0)


_sc_aggx = functools.partial(
    pl.kernel,
    out_type=jax.ShapeDtypeStruct((_NC, _NP, _D), _F32),
    mesh=_SC_MESH,
    compiler_params=_SC_PARAMS,
    scratch_types=[
        pltpu.VMEM((_EPT,), jnp.int32),
        pltpu.VMEM((_CH,), jnp.int32),
        pltpu.VMEM((_CH,), jnp.int32),
        pltpu.VMEM((_CH,), jnp.int32),
        pltpu.VMEM((_CH, _D), _F32),
        pltpu.VMEM((_CH, _D), _F32),
        pltpu.VMEM((_CH, _D), _F32),
        pltpu.VMEM_SHARED((_NP, _D), _F32),
        pltpu.SemaphoreType.DMA,
        pltpu.SemaphoreType.DMA,
        pltpu.SemaphoreType.DMA,
        pltpu.SemaphoreType.DMA,
        pltpu.SemaphoreType.DMA,
        pltpu.SemaphoreType.DMA,
    ],
)(_sca_body)


def _scb_body(ea_hbm, dst_hbm, aggx_hbm, agge_out,
              d0, d1, d2, d3, e0, e1, e2, e3, agge_sh,
              sd0, sd1, sd2, sd3, se0, se1, se2, se3):
    del aggx_hbm
    cc = lax.axis_index("c")
    ss = lax.axis_index("s")
    eb = (cc * _NS + ss) * _EPT
    rb = ss * _RPT
    sets = ((d0, e0, sd0, se0), (d1, e1, sd1, se1),
            (d2, e2, sd2, se2), (d3, e3, sd3, se3))

    def _ze(i, _):
        e0[i, :] = jnp.zeros((16,), _F32)
        return 0

    lax.fori_loop(0, _CH, _ze, 0)

    def _zs(k, _):
        pltpu.sync_copy(e0, agge_sh.at[pl.ds(rb + k * _CH, _CH)])
        return 0

    lax.fori_loop(0, _NZ, _zs, 0)
    plsc.subcore_barrier()

    def _start(c, k):
        d, e, sd, se = sets[k]
        pltpu.async_copy(dst_hbm.at[pl.ds(eb + c * _CH, _CH)], d, sd)
        pltpu.async_copy(ea_hbm.at[pl.ds(eb + c * _CH, _CH)], e, se)

    def _drain(k):
        d, e, sd, se = sets[k]
        pltpu.make_async_copy(ea_hbm.at[pl.ds(0, _CH)], e, se).wait()
        pltpu.make_async_copy(dst_hbm.at[pl.ds(0, _CH)], d, sd).wait()
        pltpu.sync_copy(e, agge_sh.at[d], add=True)

    for k in range(4):
        _start(k, k)

    def _body(q, _):
        c0 = 4 * q
        for k in range(4):
            _drain(k)

            @pl.when(c0 + k + 4 < _NCH)
            def _(c=c0 + k + 4, k=k):
                _start(c, k)

        return 0

    lax.fori_loop(0, _NCH // 4, _body, 0)
    _drain(0)
    plsc.subcore_barrier()

    def _wb(k, _):
        r = rb + k * _CH
        pltpu.sync_copy(agge_sh.at[pl.ds(r, _CH)], e0)
        pltpu.sync_copy(e0, agge_out.at[cc, pl.ds(r, _CH)])
        return 0

    lax.fori_loop(0, _NZ, _wb, 0)


_sc_agge = functools.partial(
    pl.kernel,
    out_type=jax.ShapeDtypeStruct((_NC, _NP, _DE), _F32),
    mesh=_SC_MESH,
    compiler_params=_SC_PARAMS,
    scratch_types=[
        pltpu.VMEM((_CH,), jnp.int32),
        pltpu.VMEM((_CH,), jnp.int32),
        pltpu.VMEM((_CH,), jnp.int32),
        pltpu.VMEM((_CH,), jnp.int32),
        pltpu.VMEM((_CH, _DE), _F32),
        pltpu.VMEM((_CH, _DE), _F32),
        pltpu.VMEM((_CH, _DE), _F32),
        pltpu.VMEM((_CH, _DE), _F32),
        pltpu.VMEM_SHARED((_NP, _DE), _F32),
        pltpu.SemaphoreType.DMA,
        pltpu.SemaphoreType.DMA,
        pltpu.SemaphoreType.DMA,
        pltpu.SemaphoreType.DMA,
        pltpu.SemaphoreType.DMA,
        pltpu.SemaphoreType.DMA,
        pltpu.SemaphoreType.DMA,
        pltpu.SemaphoreType.DMA,
    ],
)(_scb_body)


def _tc_body(xr, a0r, a1r, e0r, e1r, br, wsr, wmr, wer, bmr, wpr, bpr,
             outr, gsumr, cntr):
    i = pl.program_id(0)

    @pl.when(i == 0)
    def _init():
        gsumr[...] = jnp.zeros_like(gsumr)
        cntr[...] = jnp.zeros_like(cntr)

    h = (jnp.dot(xr[...], wsr[...], precision=_HI, preferred_element_type=_F32)
         + jnp.dot(a0r[0] + a1r[0], wmr[...], precision=_HI,
                   preferred_element_type=_F32)
         + jnp.dot(e0r[0] + e1r[0], wer[...], precision=_HI,
                   preferred_element_type=_F32)
         + bmr[...])
    h = jnp.where(h > 0, h, 0.01 * h)

    oht = (br[0] == lax.broadcasted_iota(jnp.int32, (_G, _R), 0)).astype(_F32)
    gsumr[...] += jnp.dot(oht, h, precision=_HI, preferred_element_type=_F32)
    cntr[...] += jnp.dot(oht, jnp.ones((_R, _D), _F32), precision=_HI,
                         preferred_element_type=_F32)

    @pl.when(i == _NB - 1)
    def _fin():
        gmean = gsumr[...] / jnp.maximum(cntr[...], 1.0)
        n2 = jnp.sum(gmean * gmean, axis=1, keepdims=True)
        nrm = jnp.maximum(jnp.sqrt(n2), 1e-12)
        embs = (gmean / nrm).astype(jnp.bfloat16).astype(_F32)
        wp16 = wpr[...].astype(jnp.bfloat16).astype(_F32)
        outr[...] = jnp.sum(embs * wp16, axis=1, keepdims=True) + bpr[...]


_tc_head = pl.pallas_call(
    _tc_body,
    grid=(_NB,),
    in_specs=[
        pl.BlockSpec((_R, _D), lambda i: (i, 0)),
        pl.BlockSpec((1, _R, _D), lambda i: (0, i, 0)),
        pl.BlockSpec((1, _R, _D), lambda i: (1, i, 0)),
        pl.BlockSpec((1, _R, _DE), lambda i: (0, i, 0)),
        pl.BlockSpec((1, _R, _DE), lambda i: (1, i, 0)),
        pl.BlockSpec((1, 1, _R), lambda i: (i, 0, 0)),
        pl.BlockSpec((_D, _H), lambda i: (0, 0)),
        pl.BlockSpec((_D, _H), lambda i: (0, 0)),
        pl.BlockSpec((_DE, _H), lambda i: (0, 0)),
        pl.BlockSpec((1, _H), lambda i: (0, 0)),
        pl.BlockSpec((1, _H), lambda i: (0, 0)),
        pl.BlockSpec((1, 1), lambda i: (0, 0)),
    ],
    out_specs=pl.BlockSpec((_G, 1), lambda i: (0, 0)),
    out_shape=jax.ShapeDtypeStruct((_G, 1), _F32),
    scratch_shapes=[
        pltpu.VMEM((_G, _D), _F32),
        pltpu.VMEM((_G, _D), _F32),
    ],
)


def kernel(x, edge_index, edge_attr, batch, W_self, W_msg, W_edge, b_msg,
           Wp, bp):
    src, dst = _prep(edge_index)
    aggx = _sc_aggx(x, src, dst)
    agge = _sc_agge(edge_attr, dst, aggx)
    return _tc_head(x, aggx, aggx, agge, agge, batch.reshape(_NB, 1, _R),
                    W_self, W_msg, W_edge, b_msg.reshape(1, _H),
                    Wp.reshape(1, _H), bp.reshape(1, 1))
